# baseline (speedup 1.0000x reference)
import jax
import jax.numpy as jnp
from jax import lax
from jax.experimental import pallas as pl
from jax.experimental.pallas import tpu as pltpu
from jax.experimental.pallas import tpu_sc as plsc

P = 16
EPS = 1e-07
NSEG = 16384
PB = P * P
NBINS = NSEG * PB
NPIX = 8 * 512 * 512

CHUNK = 1 << 20
CHUNK_SHIFT = 20
N_PASS = 2

N_SUBCORES = 16
PIX_PER_TILE = NPIX // N_SUBCORES
W = 4096
WR = W // 512
NW = PIX_PER_TILE // W
NB = 2


def _hist_body(grad_hbm, seg_hbm, hist_hbm, chunk_sh, gy_v, gx_v, seg_v,
               idx_v, val_v, zero_v, load_sems, scat_sems, zero_sem):
    c = lax.axis_index("c")
    s = lax.axis_index("s")

    zeros16 = jnp.zeros((16,), jnp.float32)

    @pl.loop(0, zero_v.shape[0] // 16)
    def _fill_zero(i):
        zero_v[pl.ds(i * 16, 16)] = zeros16

    zlen = zero_v.shape[0]
    slice_per_tile = CHUNK // N_SUBCORES
    batch = s // 2
    row_base = (s % 2) * 256

    def _issue_loads(w, b):
        r0 = row_base + w * WR
        return [
            pltpu.async_copy(grad_hbm.at[batch, 0, pl.ds(r0, WR)], gy_v[b],
                             load_sems[b]),
            pltpu.async_copy(grad_hbm.at[batch, 1, pl.ds(r0, WR)], gx_v[b],
                             load_sems[b]),
            pltpu.async_copy(seg_hbm.at[batch, pl.ds(r0, WR)], seg_v[b],
                             load_sems[b]),
        ]

    for p in range(N_PASS):
        chunk_id = p * 2 + c

        def _remap_window(b, _chunk_id=chunk_id):
            gy2 = gy_v[b]
            gx2 = gx_v[b]
            sg2 = seg_v[b]
            ib = idx_v[b]
            vb = val_v[b]
            lo = EPS - 1.0
            hi = 1.0 - EPS

            @pl.loop(0, W // 16, unroll=2)
            def _remap(i):
                r = i // 32
                col = (i % 32) * 16
                gy = jnp.clip(gy2[r, pl.ds(col, 16)], lo, hi)
                gx = jnp.clip(gx2[r, pl.ds(col, 16)], lo, hi)
                yi = ((gy + 1.0) * (P / 2.0)).astype(jnp.int32)
                xi = ((gx + 1.0) * (P / 2.0)).astype(jnp.int32)
                pos = sg2[r, pl.ds(col, 16)] * PB + yi * P + xi
                ok = (pos >> CHUNK_SHIFT) == _chunk_id
                ib[pl.ds(i * 16, 16)] = pos & (CHUNK - 1)
                vb[pl.ds(i * 16, 16)] = jnp.where(ok, 1.0, 0.0)

        loads = [None] * NW
        scats = [None] * NW
        loads[0] = _issue_loads(0, 0)
        zeros = [
            pltpu.async_copy(
                zero_v,
                chunk_sh.at[pl.ds(s * slice_per_tile + j * zlen, zlen)],
                zero_sem)
            for j in range(slice_per_tile // zlen)
        ]
        for d in loads[0]:
            d.wait()
        _remap_window(0)
        for z in zeros:
            z.wait()

        plsc.subcore_barrier()

        for w in range(NW):
            b = w % NB
            if w >= 1:
                for d in loads[w]:
                    d.wait()
                _remap_window(b)
                scats[w - 1].wait()
            if w + 1 < NW:
                loads[w + 1] = _issue_loads(w + 1, (w + 1) % NB)

            scats[w] = pltpu.async_copy(val_v[b], chunk_sh.at[idx_v[b]],
                                        scat_sems[b], add=True)

        scats[NW - 1].wait()

        plsc.subcore_barrier()

        base = chunk_id * CHUNK
        pltpu.sync_copy(chunk_sh.at[pl.ds(s * slice_per_tile, slice_per_tile)],
                        hist_hbm.at[pl.ds(base + s * slice_per_tile,
                                          slice_per_tile)])


def _scatter_hist(grad, seg):
    kern = pl.kernel(
        _hist_body,
        out_type=jax.ShapeDtypeStruct((NBINS,), jnp.float32),
        mesh=plsc.VectorSubcoreMesh(core_axis_name="c", subcore_axis_name="s"),
        compiler_params=pltpu.CompilerParams(needs_layout_passes=False),
        scratch_types=[
            pltpu.VMEM_SHARED((CHUNK,), jnp.float32),
            [pltpu.VMEM((WR, 512), jnp.float32) for _ in range(NB)],
            [pltpu.VMEM((WR, 512), jnp.float32) for _ in range(NB)],
            [pltpu.VMEM((WR, 512), jnp.int32) for _ in range(NB)],
            [pltpu.VMEM((W,), jnp.int32) for _ in range(NB)],
            [pltpu.VMEM((W,), jnp.float32) for _ in range(NB)],
            pltpu.VMEM((8192,), jnp.float32),
            [pltpu.SemaphoreType.DMA for _ in range(NB)],
            [pltpu.SemaphoreType.DMA for _ in range(NB)],
            pltpu.SemaphoreType.DMA,
        ],
    )
    return kern(grad, seg)


def _final_body(hist_ref, out_ref):
    h = hist_ref[...].reshape(2048, PB)
    sizes = jnp.sum(h, axis=1, keepdims=True)
    out_ref[...] = h / (sizes * ((P / 32.0) ** 2))


def _finalize(hist):
    return pl.pallas_call(
        _final_body,
        grid=(8,),
        in_specs=[pl.BlockSpec((2048 * PB,), lambda i: (i,))],
        out_specs=pl.BlockSpec((2048, PB), lambda i: (i, 0)),
        out_shape=jax.ShapeDtypeStruct((NSEG, PB), jnp.float32),
    )(hist)


def kernel(grad, seg, fV, nV):
    hist = _scatter_hist(grad, seg.astype(jnp.int32))
    out = _finalize(hist)
    return out.reshape(NSEG, 1, P, P)


# speedup vs baseline: 1.5991x; 1.5991x over previous
import jax
import jax.numpy as jnp
from jax import lax
from jax.experimental import pallas as pl
from jax.experimental.pallas import tpu as pltpu
from jax.experimental.pallas import tpu_sc as plsc

P = 16
EPS = 1e-07
NSEG = 16384
PB = P * P
NBINS = NSEG * PB
NPIX = 8 * 512 * 512

CHUNK = 1 << 20
N_PASS = 2

N_SUBCORES = 16
PIX_PER_TILE = NPIX // N_SUBCORES
W = 8192
NW = PIX_PER_TILE // W


def _pos_body(grad_ref, seg_ref, pos_ref):
    g = grad_ref[0]
    seg = seg_ref[0]
    lo = EPS - 1.0
    hi = 1.0 - EPS
    gy = jnp.clip(g[0], lo, hi)
    gx = jnp.clip(g[1], lo, hi)
    yi = ((gy + 1.0) * (P / 2.0)).astype(jnp.int32)
    xi = ((gx + 1.0) * (P / 2.0)).astype(jnp.int32)
    pos_ref[...] = (seg * PB + yi * P + xi).reshape(512 * 512)


def _compute_pos(grad, seg):
    return pl.pallas_call(
        _pos_body,
        grid=(8,),
        in_specs=[
            pl.BlockSpec((1, 2, 512, 512), lambda i: (i, 0, 0, 0)),
            pl.BlockSpec((1, 512, 512), lambda i: (i, 0, 0)),
        ],
        out_specs=pl.BlockSpec((512 * 512,), lambda i: (i,)),
        out_shape=jax.ShapeDtypeStruct((NPIX,), jnp.int32),
    )(grad, seg)


CHUNK_SHIFT = 20


def _hist_body(pos_hbm, hist_hbm, chunk_sh, idx_v, val_v, zero_v,
               load_sems, scat_sems, zero_sem):
    c = lax.axis_index("c")
    s = lax.axis_index("s")

    zeros16 = jnp.zeros((16,), jnp.float32)

    @pl.loop(0, zero_v.shape[0] // 16)
    def _fill_zero(i):
        zero_v[pl.ds(i * 16, 16)] = zeros16

    zlen = zero_v.shape[0]
    slice_per_tile = CHUNK // N_SUBCORES
    pix_base = s * PIX_PER_TILE
    NB = len(idx_v)

    for p in range(N_PASS):
        chunk_id = p * 2 + c
        base = chunk_id * CHUNK

        def _remap_window(b, _chunk_id=chunk_id):
            ib = idx_v[b]
            vb = val_v[b]

            @pl.loop(0, W // 16, unroll=4)
            def _remap(i):
                idx = ib[pl.ds(i * 16, 16)]
                ok = (idx >> CHUNK_SHIFT) == _chunk_id
                local = idx & (CHUNK - 1)
                ib[pl.ds(i * 16, 16)] = local
                vb[pl.ds(i * 16, 16)] = jnp.where(ok, 1.0, 0.0)

        loads = [None] * NW
        scats = [None] * NW
        loads[0] = pltpu.async_copy(pos_hbm.at[pl.ds(pix_base, W)],
                                    idx_v[0], load_sems[0])
        loads[1] = pltpu.async_copy(pos_hbm.at[pl.ds(pix_base + W, W)],
                                    idx_v[1], load_sems[1])
        zeros = [
            pltpu.async_copy(
                zero_v,
                chunk_sh.at[pl.ds(s * slice_per_tile + j * zlen, zlen)],
                zero_sem)
            for j in range(slice_per_tile // zlen)
        ]
        loads[0].wait()
        _remap_window(0)
        for z in zeros:
            z.wait()

        plsc.subcore_barrier()

        for w in range(NW):
            b = w % NB
            if w >= 1:
                loads[w].wait()
                if w >= 2:
                    scats[w - 2].wait()
                if w + 1 < NW:
                    nb = (w + 1) % NB
                    loads[w + 1] = pltpu.async_copy(
                        pos_hbm.at[pl.ds(pix_base + (w + 1) * W, W)],
                        idx_v[nb], load_sems[nb])
                _remap_window(b)

            scats[w] = pltpu.async_copy(val_v[b], chunk_sh.at[idx_v[b]],
                                        scat_sems[b], add=True)

        scats[NW - 2].wait()
        scats[NW - 1].wait()

        plsc.subcore_barrier()

        pltpu.sync_copy(chunk_sh.at[pl.ds(s * slice_per_tile, slice_per_tile)],
                        hist_hbm.at[pl.ds(base + s * slice_per_tile,
                                          slice_per_tile)])


def _scatter_hist(pos_flat):
    kern = pl.kernel(
        _hist_body,
        out_type=jax.ShapeDtypeStruct((NBINS,), jnp.float32),
        mesh=plsc.VectorSubcoreMesh(core_axis_name="c", subcore_axis_name="s"),
        compiler_params=pltpu.CompilerParams(needs_layout_passes=False),
        scratch_types=[
            pltpu.VMEM_SHARED((CHUNK,), jnp.float32),
            [pltpu.VMEM((W,), jnp.int32) for _ in range(3)],
            [pltpu.VMEM((W,), jnp.float32) for _ in range(3)],
            pltpu.VMEM((8192,), jnp.float32),
            [pltpu.SemaphoreType.DMA for _ in range(3)],
            [pltpu.SemaphoreType.DMA for _ in range(3)],
            pltpu.SemaphoreType.DMA,
        ],
    )
    return kern(pos_flat)


def _final_body(hist_ref, out_ref):
    h = hist_ref[...].reshape(2048, PB)
    sizes = jnp.sum(h, axis=1, keepdims=True)
    out_ref[...] = h / (sizes * ((P / 32.0) ** 2))


def _finalize(hist):
    return pl.pallas_call(
        _final_body,
        grid=(8,),
        in_specs=[pl.BlockSpec((2048 * PB,), lambda i: (i,))],
        out_specs=pl.BlockSpec((2048, PB), lambda i: (i, 0)),
        out_shape=jax.ShapeDtypeStruct((NSEG, PB), jnp.float32),
    )(hist)


def kernel(grad, seg, fV, nV):
    pos = _compute_pos(grad, seg.astype(jnp.int32))
    hist = _scatter_hist(pos)
    out = _finalize(hist)
    return out.reshape(NSEG, 1, P, P)
